# Initial kernel scaffold; baseline (speedup 1.0000x reference)
import jax
import jax.numpy as jnp
from jax.experimental import pallas as pl


def kernel(x, emb, W, b):
    raise NotImplementedError("write your pallas kernel here")


# speedup vs baseline: 2.3629x; 2.3629x over previous
import functools

import jax
import jax.numpy as jnp
from jax import lax
from jax.experimental import pallas as pl
from jax.experimental.pallas import tpu as pltpu
from jax.experimental.pallas import tpu_sc as plsc

D = 64
NC, NS = 2, 16
NW = NC * NS
CHUNK = 128
NBUF = 4
L = 16


def _pair_table(emb, W, b):
    def body(emb_ref, w_ref, b_ref, out_ref):
        t = lax.dot_general(
            emb_ref[...], w_ref[...], (((1,), (1,)), ((), ())),
            preferred_element_type=jnp.float32) + b_ref[...]
        left = jnp.broadcast_to(t[:, None, :], (D, D, D))
        right = jnp.broadcast_to(t[None, :, :], (D, D, D))
        out_ref[...] = jnp.concatenate([left, right], axis=-1)

    return pl.pallas_call(
        body,
        out_shape=jax.ShapeDtypeStruct((D, D, 2 * D), jnp.float32),
    )(emb, W, b.reshape(1, D))


@functools.partial(jax.jit, static_argnames=("n_chunks",))
def _sc_gather(table2, idx_ev, idx_od, n_chunks):
    n_pairs_w = n_chunks * CHUNK
    n_rows = NW * n_pairs_w
    mesh = plsc.VectorSubcoreMesh(
        core_axis_name="c", subcore_axis_name="s",
        num_cores=NC, num_subcores=NS)

    @functools.partial(
        pl.kernel,
        out_type=jax.ShapeDtypeStruct((n_rows, 2 * D), jnp.float32),
        mesh=mesh,
        scratch_types=[
            pltpu.VMEM((n_pairs_w,), jnp.int32),
            pltpu.VMEM((n_pairs_w,), jnp.int32),
            pltpu.VMEM((n_pairs_w,), jnp.int32),
            pltpu.VMEM((NBUF, CHUNK, 2 * D), jnp.float32),
            pltpu.SemaphoreType.DMA((NBUF,)),
            pltpu.SemaphoreType.DMA((NBUF,)),
        ],
    )
    def k(tab_hbm, ev_hbm, od_hbm, out_hbm, ev_v, od_v, pidx_v, bufs,
          gsem, wsem):
        table_hbm = tab_hbm
        wid = lax.axis_index("s") * NC + lax.axis_index("c")
        base = wid * n_pairs_w
        pltpu.sync_copy(ev_hbm.at[wid], ev_v)
        pltpu.sync_copy(od_hbm.at[wid], od_v)

        def pair_body(j, carry):
            s = pl.ds(j * L, L)
            pidx_v[s] = ev_v[s] * D + od_v[s]
            return carry

        lax.fori_loop(0, n_pairs_w // L, pair_body, 0)

        def start_gather(ch, p):
            pltpu.async_copy(
                table_hbm.at[pidx_v.at[pl.ds(ch * CHUNK, CHUNK)]],
                bufs.at[p], gsem.at[p])

        def wait_gather(ch, p):
            pltpu.make_async_copy(
                table_hbm.at[pidx_v.at[pl.ds(ch * CHUNK, CHUNK)]],
                bufs.at[p], gsem.at[p]).wait()

        def out_slice(ch):
            return out_hbm.at[pl.ds(base + ch * CHUNK, CHUNK)]

        for p in range(NBUF):
            start_gather(p, p)

        def body(i, carry):
            for p in range(NBUF):
                ch = i * NBUF + p
                wait_gather(ch, p)
                pltpu.async_copy(bufs.at[p], out_slice(ch), wsem.at[p])
            for p in range(NBUF):
                ch = i * NBUF + p
                pltpu.make_async_copy(
                    bufs.at[p], out_slice(ch), wsem.at[p]).wait()
                start_gather((i + 1) * NBUF + p, p)
            return carry

        lax.fori_loop(0, n_chunks // NBUF - 1, body, 0)

        last = n_chunks - NBUF
        for p in range(NBUF):
            ch = last + p
            wait_gather(ch, p)
            pltpu.async_copy(bufs.at[p], out_slice(ch), wsem.at[p])
        for p in range(NBUF):
            ch = last + p
            pltpu.make_async_copy(
                bufs.at[p], out_slice(ch), wsem.at[p]).wait()

    return k(table2, idx_ev, idx_od)


def kernel(x, emb, W, b):
    bsz, seq = x.shape
    n = bsz * seq
    n_chunks = n // (NW * 2 * CHUNK)
    pairs = x.reshape(NW, n // (2 * NW), 2).astype(jnp.int32)
    table2 = _pair_table(emb, W, b).reshape(D * D, 2 * D)
    out = _sc_gather(table2, pairs[:, :, 0], pairs[:, :, 1], n_chunks)
    return out.reshape(bsz, seq, D)


# speedup vs baseline: 3.1143x; 1.3180x over previous
import functools

import jax
import jax.numpy as jnp
from jax import lax
from jax.experimental import pallas as pl
from jax.experimental.pallas import tpu as pltpu
from jax.experimental.pallas import tpu_sc as plsc

D = 64
NC, NS = 2, 16
NW = NC * NS
CHUNK = 128
NBUF = 4
L = 16


def _pair_table(emb, W, b):
    def body(emb_ref, w_ref, b_ref, out_ref):
        t = lax.dot_general(
            emb_ref[...], w_ref[...], (((1,), (1,)), ((), ())),
            preferred_element_type=jnp.float32) + b_ref[...]
        left = jnp.broadcast_to(t[:, None, :], (D, D, D))
        right = jnp.broadcast_to(t[None, :, :], (D, D, D))
        out_ref[...] = jnp.concatenate([left, right], axis=-1)

    return pl.pallas_call(
        body,
        out_shape=jax.ShapeDtypeStruct((D, D, 2 * D), jnp.float32),
    )(emb, W, b.reshape(1, D))


@functools.partial(jax.jit, static_argnames=("n_chunks",))
def _sc_gather(table2, idx_ev, idx_od, n_chunks):
    n_pairs_w = n_chunks * CHUNK
    n_rows = NW * n_pairs_w
    mesh = plsc.VectorSubcoreMesh(
        core_axis_name="c", subcore_axis_name="s",
        num_cores=NC, num_subcores=NS)

    @functools.partial(
        pl.kernel,
        out_type=jax.ShapeDtypeStruct((n_rows, 2 * D), jnp.float32),
        mesh=mesh,
        scratch_types=[
            pltpu.VMEM((n_pairs_w,), jnp.int32),
            pltpu.VMEM((n_pairs_w,), jnp.int32),
            pltpu.VMEM((n_pairs_w,), jnp.int32),
            pltpu.VMEM((NBUF, CHUNK, 2 * D), jnp.float32),
            pltpu.SemaphoreType.DMA((NBUF,)),
            pltpu.SemaphoreType.DMA((NBUF,)),
        ],
    )
    def k(table_hbm, ev_hbm, od_hbm, out_hbm, ev_v, od_v, pidx_v, bufs,
          gsem, wsem):
        wid = lax.axis_index("s") * NC + lax.axis_index("c")
        base = wid * n_pairs_w
        pltpu.sync_copy(ev_hbm.at[pl.ds(base, n_pairs_w)], ev_v)
        pltpu.sync_copy(od_hbm.at[pl.ds(base, n_pairs_w)], od_v)

        def pair_body(j, carry):
            s = pl.ds(j * L, L)
            pidx_v[s] = ev_v[s] * D + od_v[s]
            return carry

        lax.fori_loop(0, n_pairs_w // L, pair_body, 0)

        def start_gather(ch, p):
            pltpu.async_copy(
                table_hbm.at[pidx_v.at[pl.ds(ch * CHUNK, CHUNK)]],
                bufs.at[p], gsem.at[p])

        def wait_gather(ch, p):
            pltpu.make_async_copy(
                table_hbm.at[pidx_v.at[pl.ds(ch * CHUNK, CHUNK)]],
                bufs.at[p], gsem.at[p]).wait()

        def out_slice(ch):
            return out_hbm.at[pl.ds(base + ch * CHUNK, CHUNK)]

        for p in range(NBUF):
            start_gather(p, p)

        def body(i, carry):
            for p in range(NBUF):
                ch = i * NBUF + p
                wait_gather(ch, p)
                pltpu.async_copy(bufs.at[p], out_slice(ch), wsem.at[p])
            for p in range(NBUF):
                ch = i * NBUF + p
                pltpu.make_async_copy(
                    bufs.at[p], out_slice(ch), wsem.at[p]).wait()
                start_gather((i + 1) * NBUF + p, p)
            return carry

        lax.fori_loop(0, n_chunks // NBUF - 1, body, 0)

        last = n_chunks - NBUF
        for p in range(NBUF):
            ch = last + p
            wait_gather(ch, p)
            pltpu.async_copy(bufs.at[p], out_slice(ch), wsem.at[p])
        for p in range(NBUF):
            ch = last + p
            pltpu.make_async_copy(
                bufs.at[p], out_slice(ch), wsem.at[p]).wait()

    return k(table2, idx_ev, idx_od)


def kernel(x, emb, W, b):
    bsz, seq = x.shape
    n = bsz * seq
    n_chunks = n // (NW * 2 * CHUNK)
    pairs = x.reshape(n // 2, 2).astype(jnp.int32)
    table2 = _pair_table(emb, W, b).reshape(D * D, 2 * D)
    out = _sc_gather(table2, pairs[:, 0], pairs[:, 1], n_chunks)
    return out.reshape(bsz, seq, D)


# speedup vs baseline: 3.5165x; 1.1291x over previous
import functools

import jax
import jax.numpy as jnp
from jax import lax
from jax.experimental import pallas as pl
from jax.experimental.pallas import tpu as pltpu
from jax.experimental.pallas import tpu_sc as plsc

D = 64
NC, NS = 2, 16
NW = NC * NS
CHUNK = 128
NBUF = 4
L = 16


def _pair_table(emb, W, b):
    def body(emb_ref, w_ref, b_ref, out_ref):
        t = lax.dot_general(
            emb_ref[...], w_ref[...], (((1,), (1,)), ((), ())),
            preferred_element_type=jnp.float32) + b_ref[...]
        left = jnp.broadcast_to(t[:, None, :], (D, D, D))
        right = jnp.broadcast_to(t[None, :, :], (D, D, D))
        out_ref[...] = jnp.concatenate([left, right], axis=-1)

    return pl.pallas_call(
        body,
        out_shape=jax.ShapeDtypeStruct((D, D, 2 * D), jnp.float32),
    )(emb, W, b.reshape(1, D))


@functools.partial(jax.jit, static_argnames=("n_chunks",))
def _sc_gather(table2, packed, n_chunks):
    n_pairs_w = n_chunks * CHUNK
    n_rows = NW * n_pairs_w
    mesh = plsc.VectorSubcoreMesh(
        core_axis_name="c", subcore_axis_name="s",
        num_cores=NC, num_subcores=NS)

    @functools.partial(
        pl.kernel,
        out_type=jax.ShapeDtypeStruct((n_rows, 2 * D), jnp.float32),
        mesh=mesh,
        scratch_types=[
            pltpu.VMEM((n_pairs_w,), jnp.int32),
            pltpu.VMEM((n_pairs_w,), jnp.int32),
            pltpu.VMEM((NBUF, CHUNK, 2 * D), jnp.float32),
            pltpu.SemaphoreType.DMA((NBUF,)),
            pltpu.SemaphoreType.DMA((NBUF,)),
        ],
    )
    def k(table_hbm, pk_hbm, out_hbm, pk_v, pidx_v, bufs, gsem, wsem):
        wid = lax.axis_index("s") * NC + lax.axis_index("c")
        base = wid * n_pairs_w
        pltpu.sync_copy(pk_hbm.at[pl.ds(base, n_pairs_w)], pk_v)

        def pair_body(j, carry):
            s = pl.ds(j * L, L)
            v = pk_v[s]
            pidx_v[s] = (v & 0xFFFF) * D + (v >> 16)
            return carry

        lax.fori_loop(0, n_pairs_w // L, pair_body, 0)

        def start_gather(ch, p):
            pltpu.async_copy(
                table_hbm.at[pidx_v.at[pl.ds(ch * CHUNK, CHUNK)]],
                bufs.at[p], gsem.at[p])

        def wait_gather(ch, p):
            pltpu.make_async_copy(
                table_hbm.at[pidx_v.at[pl.ds(ch * CHUNK, CHUNK)]],
                bufs.at[p], gsem.at[p]).wait()

        def out_slice(ch):
            return out_hbm.at[pl.ds(base + ch * CHUNK, CHUNK)]

        for p in range(NBUF):
            start_gather(p, p)

        def body(i, carry):
            for p in range(NBUF):
                ch = i * NBUF + p
                wait_gather(ch, p)
                pltpu.async_copy(bufs.at[p], out_slice(ch), wsem.at[p])
            for p in range(NBUF):
                ch = i * NBUF + p
                pltpu.make_async_copy(
                    bufs.at[p], out_slice(ch), wsem.at[p]).wait()
                start_gather((i + 1) * NBUF + p, p)
            return carry

        lax.fori_loop(0, n_chunks // NBUF - 1, body, 0)

        last = n_chunks - NBUF
        for p in range(NBUF):
            ch = last + p
            wait_gather(ch, p)
            pltpu.async_copy(bufs.at[p], out_slice(ch), wsem.at[p])
        for p in range(NBUF):
            ch = last + p
            pltpu.make_async_copy(
                bufs.at[p], out_slice(ch), wsem.at[p]).wait()

    return k(table2, packed)


def kernel(x, emb, W, b):
    bsz, seq = x.shape
    n = bsz * seq
    n_chunks = n // (NW * 2 * CHUNK)
    packed = lax.bitcast_convert_type(
        x.reshape(n // 2, 2).astype(jnp.int16), jnp.int32)
    table2 = _pair_table(emb, W, b).reshape(D * D, 2 * D)
    out = _sc_gather(table2, packed, n_chunks)
    return out.reshape(bsz, seq, D)
